# baseline (device time: 22256 ns/iter reference)
import jax
import jax.numpy as jnp
from jax import lax
from jax.experimental import pallas as pl
from jax.experimental.pallas import tpu as pltpu

N_DEV = 8
PAD = 32


def kernel(A, B):
    m_per, k = A.shape
    _, n = B.shape

    def body(a_ref, b_ref, out_ref, q_buf, comm_q, send_sems, recv_sems):
        my = lax.axis_index("i")


        a = a_ref[...]
        chunk_max = jnp.maximum(
            jnp.max(jnp.abs(a), axis=(0, 1), keepdims=True), 1e-30
        )
        q_buf[:m_per, :] = jnp.round(a * (127.0 / chunk_max)).astype(jnp.int8)
        s = chunk_max * (1.0 / 127.0)
        s_exp = jnp.floor(jnp.log2(s))
        s_man = jnp.round(s * jnp.exp2(-s_exp) * 64.0) - 64.0
        q_buf[m_per : m_per + 1, 0:1] = s_exp.astype(jnp.int8)
        q_buf[m_per : m_per + 1, 1:2] = s_man.astype(jnp.int8)

        rdmas = []
        for j in range(1, N_DEV):
            dst = (my + j) % N_DEV
            rdma = pltpu.make_async_remote_copy(
                src_ref=q_buf,
                dst_ref=comm_q.at[my],
                send_sem=send_sems.at[j],
                recv_sem=recv_sems.at[my],
                device_id=(dst,),
                device_id_type=pl.DeviceIdType.MESH,
            )
            rdma.start()
            rdmas.append(rdma)

        b_bf = b_ref[...].astype(jnp.bfloat16)
        out_ref[pl.ds(my * m_per, m_per), :] = jnp.dot(
            a.astype(jnp.bfloat16), b_bf, preferred_element_type=jnp.float32
        )

        for j in range(1, N_DEV):
            origin = (my - j) % N_DEV
            recv = pltpu.make_async_remote_copy(
                src_ref=q_buf,
                dst_ref=comm_q.at[origin],
                send_sem=send_sems.at[j],
                recv_sem=recv_sems.at[origin],
                device_id=(origin,),
                device_id_type=pl.DeviceIdType.MESH,
            )
            recv.wait_recv()
            prod = jnp.dot(
                comm_q[origin, :m_per, :].astype(jnp.bfloat16),
                b_bf,
                preferred_element_type=jnp.float32,
            )
            s_exp = comm_q[origin, m_per : m_per + 1, 0:1].astype(jnp.float32)
            s_man = comm_q[origin, m_per : m_per + 1, 1:2].astype(jnp.float32)
            scale = (s_man + 64.0) * jnp.exp2(s_exp - 6.0)
            out_ref[pl.ds(origin * m_per, m_per), :] = prod * scale

        for rdma in rdmas:
            rdma.wait_send()

    return pl.pallas_call(
        body,
        out_shape=jax.ShapeDtypeStruct((N_DEV * m_per, n), jnp.float32),
        in_specs=[
            pl.BlockSpec(memory_space=pltpu.VMEM),
            pl.BlockSpec(memory_space=pltpu.VMEM),
        ],
        out_specs=pl.BlockSpec(memory_space=pltpu.VMEM),
        scratch_shapes=[
            pltpu.VMEM((m_per + PAD, k), jnp.int8),
            pltpu.VMEM((N_DEV, m_per + PAD, k), jnp.int8),
            pltpu.SemaphoreType.DMA((N_DEV,)),
            pltpu.SemaphoreType.DMA((N_DEV,)),
        ],
    )(A, B)


# device time: 19648 ns/iter; 1.1327x vs baseline; 1.1327x over previous
import jax
import jax.numpy as jnp
from jax import lax
from jax.experimental import pallas as pl
from jax.experimental.pallas import tpu as pltpu

N_DEV = 8
PAD = 32


def kernel(A, B):
    m_per, k = A.shape
    _, n = B.shape

    def body(a_hbm, b_hbm, out_hbm, a_vmem, b_vmem, q_buf, comm_q, c_buf,
             sem_a, sem_b, out_sems, send_sems, recv_sems):
        my = lax.axis_index("i")

        cp_a = pltpu.make_async_copy(a_hbm, a_vmem, sem_a)
        cp_a.start()
        cp_b = pltpu.make_async_copy(b_hbm, b_vmem, sem_b)
        cp_b.start()

        barrier_sem = pltpu.get_barrier_semaphore()
        for j in range(1, N_DEV):
            pl.semaphore_signal(
                barrier_sem, inc=1,
                device_id=((my + j) % N_DEV,),
                device_id_type=pl.DeviceIdType.MESH,
            )

        cp_a.wait()
        a = a_vmem[...]
        chunk_max = jnp.maximum(
            jnp.max(jnp.abs(a), axis=(0, 1), keepdims=True), 1e-30
        )
        q_buf[:m_per, :] = jnp.round(a * (127.0 / chunk_max)).astype(jnp.int8)
        s = chunk_max * (1.0 / 127.0)
        s_exp = jnp.floor(jnp.log2(s))
        s_man = jnp.round(s * jnp.exp2(-s_exp) * 64.0) - 64.0
        q_buf[m_per : m_per + 1, 0:1] = s_exp.astype(jnp.int8)
        q_buf[m_per : m_per + 1, 1:2] = s_man.astype(jnp.int8)

        pl.semaphore_wait(barrier_sem, N_DEV - 1)

        rdmas = []
        for j in range(1, N_DEV):
            dst = (my + j) % N_DEV
            rdma = pltpu.make_async_remote_copy(
                src_ref=q_buf,
                dst_ref=comm_q.at[my],
                send_sem=send_sems.at[j],
                recv_sem=recv_sems.at[my],
                device_id=(dst,),
                device_id_type=pl.DeviceIdType.MESH,
            )
            rdma.start()
            rdmas.append(rdma)

        def emit(origin, block):
            c_buf[origin, :, :] = block
            out_cp = pltpu.make_async_copy(
                c_buf.at[origin],
                out_hbm.at[pl.ds(origin * m_per, m_per), :],
                out_sems.at[origin],
            )
            out_cp.start()
            return out_cp

        cp_b.wait()
        b_bf = b_vmem[...].astype(jnp.bfloat16)
        out_cps = [emit(my, jnp.dot(
            a.astype(jnp.bfloat16), b_bf, preferred_element_type=jnp.float32
        ))]

        for j in range(1, N_DEV):
            origin = (my - j) % N_DEV
            recv = pltpu.make_async_remote_copy(
                src_ref=q_buf,
                dst_ref=comm_q.at[origin],
                send_sem=send_sems.at[j],
                recv_sem=recv_sems.at[origin],
                device_id=(origin,),
                device_id_type=pl.DeviceIdType.MESH,
            )
            recv.wait_recv()
            prod = jnp.dot(
                comm_q[origin, :m_per, :].astype(jnp.bfloat16),
                b_bf,
                preferred_element_type=jnp.float32,
            )
            s_exp = comm_q[origin, m_per : m_per + 1, 0:1].astype(jnp.float32)
            s_man = comm_q[origin, m_per : m_per + 1, 1:2].astype(jnp.float32)
            out_cps.append(emit(origin, prod * ((s_man + 64.0)
                                                * jnp.exp2(s_exp - 6.0))))

        for cp in out_cps:
            cp.wait()
        for rdma in rdmas:
            rdma.wait_send()

    return pl.pallas_call(
        body,
        out_shape=jax.ShapeDtypeStruct((N_DEV * m_per, n), jnp.float32),
        in_specs=[
            pl.BlockSpec(memory_space=pltpu.MemorySpace.HBM),
            pl.BlockSpec(memory_space=pltpu.MemorySpace.HBM),
        ],
        out_specs=pl.BlockSpec(memory_space=pltpu.MemorySpace.HBM),
        scratch_shapes=[
            pltpu.VMEM((m_per, k), jnp.float32),
            pltpu.VMEM((k, n), jnp.float32),
            pltpu.VMEM((m_per + PAD, k), jnp.int8),
            pltpu.VMEM((N_DEV, m_per + PAD, k), jnp.int8),
            pltpu.VMEM((N_DEV, m_per, n), jnp.float32),
            pltpu.SemaphoreType.DMA,
            pltpu.SemaphoreType.DMA,
            pltpu.SemaphoreType.DMA((N_DEV,)),
            pltpu.SemaphoreType.DMA((N_DEV,)),
            pltpu.SemaphoreType.DMA((N_DEV,)),
        ],
        compiler_params=pltpu.CompilerParams(collective_id=0),
    )(A, B)


# device time: 19587 ns/iter; 1.1363x vs baseline; 1.0031x over previous
import jax
import jax.numpy as jnp
from jax import lax
from jax.experimental import pallas as pl
from jax.experimental.pallas import tpu as pltpu

N_DEV = 8
PAD = 32


def kernel(A, B):
    m_per, k = A.shape
    _, n = B.shape

    def body(a_hbm, b_hbm, out_hbm, a_vmem, b_vmem, q_buf, comm_q, c_buf,
             sem_a, sem_b, out_sems, send_sems, recv_sems):
        my = lax.axis_index("i")

        cp_a = pltpu.make_async_copy(a_hbm, a_vmem, sem_a)
        cp_a.start()
        cp_b = pltpu.make_async_copy(b_hbm, b_vmem, sem_b)
        cp_b.start()

        barrier_sem = pltpu.get_barrier_semaphore()
        for j in range(1, N_DEV):
            pl.semaphore_signal(
                barrier_sem, inc=1,
                device_id=((my + j) % N_DEV,),
                device_id_type=pl.DeviceIdType.MESH,
            )

        cp_a.wait()
        a = a_vmem[...]
        chunk_max = jnp.maximum(
            jnp.max(jnp.abs(a), axis=(0, 1), keepdims=True), 1e-30
        )
        q_buf[:m_per, :] = jnp.round(a * (127.0 / chunk_max)).astype(jnp.int8)
        s = chunk_max * (1.0 / 127.0)
        s_exp = jnp.floor(jnp.log2(s))
        s_man = jnp.round(s * jnp.exp2(-s_exp) * 64.0) - 64.0
        q_buf[m_per : m_per + 1, 0:1] = s_exp.astype(jnp.int8)
        q_buf[m_per : m_per + 1, 1:2] = s_man.astype(jnp.int8)

        pl.semaphore_wait(barrier_sem, N_DEV - 1)

        rdmas = []
        for j in range(1, N_DEV):
            dst = (my + j) % N_DEV
            rdma = pltpu.make_async_remote_copy(
                src_ref=q_buf,
                dst_ref=comm_q.at[my],
                send_sem=send_sems.at[j],
                recv_sem=recv_sems.at[my],
                device_id=(dst,),
                device_id_type=pl.DeviceIdType.MESH,
            )
            rdma.start()
            rdmas.append(rdma)

        def emit(origin, block):
            c_buf[origin, :, :] = block
            out_cp = pltpu.make_async_copy(
                c_buf.at[origin],
                out_hbm.at[pl.ds(origin * m_per, m_per), :],
                out_sems.at[origin],
            )
            out_cp.start()
            return out_cp

        cp_b.wait()
        b_bf = b_vmem[...].astype(jnp.bfloat16)
        out_cps = [emit(my, jnp.dot(
            a.astype(jnp.bfloat16), b_bf, preferred_element_type=jnp.float32
        ))]

        for j in range(1, N_DEV):
            origin = (my - j) % N_DEV
            recv = pltpu.make_async_remote_copy(
                src_ref=q_buf,
                dst_ref=comm_q.at[origin],
                send_sem=send_sems.at[j],
                recv_sem=recv_sems.at[origin],
                device_id=(origin,),
                device_id_type=pl.DeviceIdType.MESH,
            )
            recv.wait_recv()
            prod = jnp.dot(
                comm_q[origin, :m_per, :].astype(jnp.bfloat16),
                b_bf,
                preferred_element_type=jnp.float32,
            )
            s_exp = comm_q[origin, m_per : m_per + 1, 0:1].astype(jnp.float32)
            s_man = comm_q[origin, m_per : m_per + 1, 1:2].astype(jnp.float32)
            out_cps.append(emit(origin, prod * ((s_man + 64.0)
                                                * jnp.exp2(s_exp - 6.0))))

        for cp in out_cps:
            cp.wait()
        for rdma in rdmas:
            rdma.wait_send()

    return pl.pallas_call(
        body,
        out_shape=jax.ShapeDtypeStruct((N_DEV * m_per, n), jnp.float32),
        in_specs=[
            pl.BlockSpec(memory_space=pl.ANY),
            pl.BlockSpec(memory_space=pl.ANY),
        ],
        out_specs=pl.BlockSpec(memory_space=pl.ANY),
        scratch_shapes=[
            pltpu.VMEM((m_per, k), jnp.float32),
            pltpu.VMEM((k, n), jnp.float32),
            pltpu.VMEM((m_per + PAD, k), jnp.int8),
            pltpu.VMEM((N_DEV, m_per + PAD, k), jnp.int8),
            pltpu.VMEM((N_DEV, m_per, n), jnp.float32),
            pltpu.SemaphoreType.DMA,
            pltpu.SemaphoreType.DMA,
            pltpu.SemaphoreType.DMA((N_DEV,)),
            pltpu.SemaphoreType.DMA((N_DEV,)),
            pltpu.SemaphoreType.DMA((N_DEV,)),
        ],
        compiler_params=pltpu.CompilerParams(collective_id=0),
    )(A, B)


# device time: 17754 ns/iter; 1.2536x vs baseline; 1.1032x over previous
import jax
import jax.numpy as jnp
from jax import lax
from jax.experimental import pallas as pl
from jax.experimental.pallas import tpu as pltpu

N_DEV = 8
PAD = 32


def kernel(A, B):
    m_per, k = A.shape
    _, n = B.shape

    def body(a_ref, b_ref, out_ref, q_buf, comm_q, send_sems, recv_sems):
        my = lax.axis_index("i")

        barrier_sem = pltpu.get_barrier_semaphore()
        for j in range(1, N_DEV):
            pl.semaphore_signal(
                barrier_sem, inc=1,
                device_id=((my + j) % N_DEV,),
                device_id_type=pl.DeviceIdType.MESH,
            )

        a = a_ref[...]
        chunk_max = jnp.maximum(
            jnp.max(jnp.abs(a), axis=(0, 1), keepdims=True), 1e-30
        )
        q_buf[:m_per, :] = jnp.round(a * (127.0 / chunk_max)).astype(jnp.int8)
        s = chunk_max * (1.0 / 127.0)
        s_exp = jnp.floor(jnp.log2(s))
        s_man = jnp.round(s * jnp.exp2(-s_exp) * 64.0) - 64.0
        q_buf[m_per : m_per + 1, 0:1] = s_exp.astype(jnp.int8)
        q_buf[m_per : m_per + 1, 1:2] = s_man.astype(jnp.int8)

        pl.semaphore_wait(barrier_sem, N_DEV - 1)

        rdmas = []
        for j in range(1, N_DEV):
            dst = (my + j) % N_DEV
            rdma = pltpu.make_async_remote_copy(
                src_ref=q_buf,
                dst_ref=comm_q.at[my],
                send_sem=send_sems.at[j],
                recv_sem=recv_sems.at[my],
                device_id=(dst,),
                device_id_type=pl.DeviceIdType.MESH,
            )
            rdma.start()
            rdmas.append(rdma)

        b_bf = b_ref[...].astype(jnp.bfloat16)
        out_ref[pl.ds(my * m_per, m_per), :] = jnp.dot(
            a.astype(jnp.bfloat16), b_bf, preferred_element_type=jnp.float32
        ).astype(jnp.bfloat16)

        for j in range(1, N_DEV):
            origin = (my - j) % N_DEV
            recv = pltpu.make_async_remote_copy(
                src_ref=q_buf,
                dst_ref=comm_q.at[origin],
                send_sem=send_sems.at[j],
                recv_sem=recv_sems.at[origin],
                device_id=(origin,),
                device_id_type=pl.DeviceIdType.MESH,
            )
            recv.wait_recv()
            prod = jnp.dot(
                comm_q[origin, :m_per, :].astype(jnp.bfloat16),
                b_bf,
                preferred_element_type=jnp.float32,
            )
            s_exp = comm_q[origin, m_per : m_per + 1, 0:1].astype(jnp.float32)
            s_man = comm_q[origin, m_per : m_per + 1, 1:2].astype(jnp.float32)
            scale = (s_man + 64.0) * jnp.exp2(s_exp - 6.0)
            out_ref[pl.ds(origin * m_per, m_per), :] = (
                prod * scale
            ).astype(jnp.bfloat16)

        for rdma in rdmas:
            rdma.wait_send()

    return pl.pallas_call(
        body,
        out_shape=jax.ShapeDtypeStruct((N_DEV * m_per, n), jnp.bfloat16),
        in_specs=[
            pl.BlockSpec(memory_space=pltpu.VMEM),
            pl.BlockSpec(memory_space=pltpu.VMEM),
        ],
        out_specs=pl.BlockSpec(memory_space=pltpu.VMEM),
        scratch_shapes=[
            pltpu.VMEM((m_per + PAD, k), jnp.int8),
            pltpu.VMEM((N_DEV, m_per + PAD, k), jnp.int8),
            pltpu.SemaphoreType.DMA((N_DEV,)),
            pltpu.SemaphoreType.DMA((N_DEV,)),
        ],
        compiler_params=pltpu.CompilerParams(collective_id=0),
    )(A, B)


# device time: 17561 ns/iter; 1.2674x vs baseline; 1.0110x over previous
import jax
import jax.numpy as jnp
from jax import lax
from jax.experimental import pallas as pl
from jax.experimental.pallas import tpu as pltpu

N_DEV = 8


def kernel(A, B):
    m_per, k = A.shape
    _, n = B.shape

    def body(a_ref, b_ref, out_ref, q_buf, s_buf, comm_q, comm_s,
             send_sems, recv_sems):
        my = lax.axis_index("i")

        barrier_sem = pltpu.get_barrier_semaphore()
        for j in range(1, N_DEV):
            pl.semaphore_signal(
                barrier_sem, inc=1,
                device_id=((my + j) % N_DEV,),
                device_id_type=pl.DeviceIdType.MESH,
            )

        a = a_ref[...]
        chunk_max = jnp.maximum(
            jnp.max(jnp.abs(a), axis=(0, 1), keepdims=True), 1e-30
        )
        q_buf[...] = jnp.round(a * (127.0 / chunk_max)).astype(jnp.int8)
        s_buf[...] = chunk_max * (1.0 / 127.0)

        pl.semaphore_wait(barrier_sem, N_DEV - 1)

        rdmas = []
        for j in range(1, N_DEV):
            dst = (my + j) % N_DEV
            for src, comm, part in ((q_buf, comm_q, 0), (s_buf, comm_s, 1)):
                rdma = pltpu.make_async_remote_copy(
                    src_ref=src,
                    dst_ref=comm.at[my],
                    send_sem=send_sems.at[j, part],
                    recv_sem=recv_sems.at[my, part],
                    device_id=(dst,),
                    device_id_type=pl.DeviceIdType.MESH,
                )
                rdma.start()
                rdmas.append(rdma)

        b_bf = b_ref[...].astype(jnp.bfloat16)
        out_ref[pl.ds(my * m_per, m_per), :] = jnp.dot(
            a.astype(jnp.bfloat16), b_bf, preferred_element_type=jnp.float32
        ).astype(jnp.bfloat16)

        for j in range(1, N_DEV):
            origin = (my - j) % N_DEV
            for comm, part in ((comm_q, 0), (comm_s, 1)):
                recv = pltpu.make_async_remote_copy(
                    src_ref=q_buf if part == 0 else s_buf,
                    dst_ref=comm.at[origin],
                    send_sem=send_sems.at[j, part],
                    recv_sem=recv_sems.at[origin, part],
                    device_id=(origin,),
                    device_id_type=pl.DeviceIdType.MESH,
                )
                recv.wait_recv()
            prod = jnp.dot(
                comm_q[origin, :, :].astype(jnp.bfloat16),
                b_bf,
                preferred_element_type=jnp.float32,
            )
            out_ref[pl.ds(origin * m_per, m_per), :] = (
                prod * comm_s[origin, :, :]
            ).astype(jnp.bfloat16)

        for rdma in rdmas:
            rdma.wait_send()

    return pl.pallas_call(
        body,
        out_shape=jax.ShapeDtypeStruct((N_DEV * m_per, n), jnp.bfloat16),
        in_specs=[
            pl.BlockSpec(memory_space=pltpu.VMEM),
            pl.BlockSpec(memory_space=pltpu.VMEM),
        ],
        out_specs=pl.BlockSpec(memory_space=pltpu.VMEM),
        scratch_shapes=[
            pltpu.VMEM((m_per, k), jnp.int8),
            pltpu.VMEM((1, 1), jnp.float32),
            pltpu.VMEM((N_DEV, m_per, k), jnp.int8),
            pltpu.VMEM((N_DEV, 1, 1), jnp.float32),
            pltpu.SemaphoreType.DMA((N_DEV, 2)),
            pltpu.SemaphoreType.DMA((N_DEV, 2)),
        ],
        compiler_params=pltpu.CompilerParams(collective_id=0),
    )(A, B)
